# baseline (device time: 21140 ns/iter reference)
import jax
import jax.numpy as jnp
from jax import lax
from jax.experimental import pallas as pl
from jax.experimental.pallas import tpu as pltpu

D_ROWS = 568
F_CHUNKS = (64, 64, 64, 64, 64, 64, 40, 32)
F_ROWS = sum(F_CHUNKS)
T_LO, T_HI = F_ROWS, D_ROWS
K = len(F_CHUNKS)
F_OFF = tuple(sum(F_CHUNKS[:i]) for i in range(K))


def kernel(x):
    m_per, n = x.shape

    def body(x_ref, out_ref, sx, rx, sy, ry, copy_sem):
        my_x = lax.axis_index("x")
        my_y = lax.axis_index("y")
        my_z = lax.axis_index("z")
        x_nbr = (1 - my_x, my_y, my_z)
        y_nbr = (my_x, 1 - my_y, my_z)

        barrier_sem = pltpu.get_barrier_semaphore()
        for nbr in (x_nbr, y_nbr):
            pl.semaphore_signal(
                barrier_sem, inc=1, device_id=nbr,
                device_id_type=pl.DeviceIdType.MESH,
            )
        pl.semaphore_wait(barrier_sem, 2)

        f_base = my_y * D_ROWS

        x_rdmas = []
        for k in range(K):
            row = f_base + F_OFF[k]
            rdma = pltpu.make_async_remote_copy(
                src_ref=x_ref.at[pl.ds(row, F_CHUNKS[k]), :],
                dst_ref=out_ref.at[pl.ds(my_x * m_per + row, F_CHUNKS[k]), :],
                send_sem=sx.at[k],
                recv_sem=rx.at[k],
                device_id=x_nbr,
                device_id_type=pl.DeviceIdType.MESH,
            )
            rdma.start()
            x_rdmas.append(rdma)
        tail = pltpu.make_async_remote_copy(
            src_ref=x_ref.at[pl.ds(T_LO, T_HI - T_LO), :],
            dst_ref=out_ref.at[pl.ds(my_x * m_per + T_LO, T_HI - T_LO), :],
            send_sem=sx.at[K],
            recv_sem=rx.at[K],
            device_id=x_nbr,
            device_id_type=pl.DeviceIdType.MESH,
        )
        tail.start()

        local_copy = pltpu.make_async_copy(
            x_ref, out_ref.at[pl.ds(my_x * m_per, m_per), :], copy_sem
        )
        local_copy.start()

        y_rdmas = []
        for k in range(K):
            x_rdmas[k].wait_recv()
            row = (1 - my_x) * m_per + f_base + F_OFF[k]
            rdma = pltpu.make_async_remote_copy(
                src_ref=out_ref.at[pl.ds(row, F_CHUNKS[k]), :],
                dst_ref=out_ref.at[pl.ds(row, F_CHUNKS[k]), :],
                send_sem=sy.at[k],
                recv_sem=ry.at[k],
                device_id=y_nbr,
                device_id_type=pl.DeviceIdType.MESH,
            )
            rdma.start()
            y_rdmas.append(rdma)

        tail.wait_recv()
        for k in range(K):
            y_rdmas[k].wait_recv()
        for k in range(K):
            x_rdmas[k].wait_send()
            y_rdmas[k].wait_send()
        tail.wait_send()
        local_copy.wait()

    return pl.pallas_call(
        body,
        out_shape=jax.ShapeDtypeStruct((2 * m_per, n), x.dtype),
        in_specs=[pl.BlockSpec(memory_space=pltpu.VMEM)],
        out_specs=pl.BlockSpec(memory_space=pltpu.VMEM),
        scratch_shapes=[
            pltpu.SemaphoreType.DMA((K + 1,)),
            pltpu.SemaphoreType.DMA((K + 1,)),
            pltpu.SemaphoreType.DMA((K,)),
            pltpu.SemaphoreType.DMA((K,)),
            pltpu.SemaphoreType.DMA,
        ],
        compiler_params=pltpu.CompilerParams(collective_id=0),
    )(x)
